# Initial kernel scaffold; baseline (speedup 1.0000x reference)
import jax
import jax.numpy as jnp
from jax.experimental import pallas as pl


def kernel(x, edge_index_r0, edge_index_r1, edge_index_r2, W1_r0, b1_r0, W2_r0, b2_r0, W1_r1, b1_r1, W2_r1, b2_r1, W1_r2, b1_r2, W2_r2, b2_r2):
    raise NotImplementedError("write your pallas kernel here")


# speedup vs baseline: 3.9934x; 3.9934x over previous
import functools

import jax
import jax.numpy as jnp
from jax import lax
from jax.experimental import pallas as pl
from jax.experimental.pallas import tpu as pltpu
from jax.experimental.pallas import tpu_sc as plsc

N = 10000
D = 128
E = 213000
NC = 2
NS = 16
NW = NC * NS
N_PAD = 10240
ROWS_PER_TILE = N_PAD // NS
EP_TILE = 6912
E_PAD = NW * EP_TILE
NCHUNK = EP_TILE // 128


def _deg_body(src_hbm, dst_hbm, out_hbm, idx_v, hist, acc_v, tmp_v, hist_sh):
    c = lax.axis_index("c")
    s = lax.axis_index("s")
    wid = s * NC + c
    ones16 = jnp.ones((16,), jnp.float32)
    zeros16 = jnp.zeros((16,), jnp.float32)

    for r in range(3):
        for side in range(2):
            ihbm = src_hbm if side == 0 else dst_hbm

            def zero_hist(i, _):
                hist[pl.ds(i * 16, 16)] = zeros16
                return 0

            lax.fori_loop(0, N_PAD // 16, zero_hist, 0)
            pltpu.sync_copy(ihbm.at[r, wid], idx_v)

            def count(j, _):
                for k in range(8):
                    vec = idx_v[j, pl.ds(k * 16, 16)]
                    plsc.addupdate_scatter(hist, [vec], ones16)
                return 0

            lax.fori_loop(0, NCHUNK, count, 0)
            pltpu.sync_copy(hist, hist_sh.at[s])
            plsc.subcore_barrier()

            def zero_acc(i, _):
                acc_v[pl.ds(i * 16, 16)] = zeros16
                return 0

            lax.fori_loop(0, ROWS_PER_TILE // 16, zero_acc, 0)
            for t in range(NS):
                pltpu.sync_copy(
                    hist_sh.at[t, pl.ds(s * ROWS_PER_TILE, ROWS_PER_TILE)], tmp_v
                )

                def accum(i, _):
                    sl = pl.ds(i * 16, 16)
                    acc_v[sl] = acc_v[sl] + tmp_v[sl]
                    return 0

                lax.fori_loop(0, ROWS_PER_TILE // 16, accum, 0)
            pltpu.sync_copy(
                acc_v, out_hbm.at[c, r, side, pl.ds(s * ROWS_PER_TILE, ROWS_PER_TILE)]
            )
            plsc.subcore_barrier()


def _agg_body(z_hbm, src_hbm, dst_hbm, out_hbm, sidx, didx, rows, zbuf, acc_sh, gsem):
    c = lax.axis_index("c")
    s = lax.axis_index("s")
    wid = s * NC + c
    zeros16 = jnp.zeros((16,), jnp.float32)

    def zero_zbuf(j, _):
        for k in range(D // 16):
            zbuf[j, pl.ds(k * 16, 16)] = zeros16
        return 0

    lax.fori_loop(0, 64, zero_zbuf, 0)

    for r in range(3):
        for q in range(ROWS_PER_TILE // 64):
            pltpu.sync_copy(zbuf, acc_sh.at[pl.ds(s * ROWS_PER_TILE + q * 64, 64)])
        pltpu.sync_copy(src_hbm.at[r, wid], sidx)
        pltpu.sync_copy(dst_hbm.at[r, wid], didx)
        plsc.subcore_barrier()

        def chunk(j, _):
            pltpu.async_copy(z_hbm.at[r].at[sidx.at[j]], rows, gsem).wait()
            pltpu.sync_copy(rows, acc_sh.at[didx.at[j]], add=True)
            return 0

        lax.fori_loop(0, NCHUNK, chunk, 0)
        plsc.subcore_barrier()
        pltpu.sync_copy(
            acc_sh.at[pl.ds(s * ROWS_PER_TILE, ROWS_PER_TILE)],
            out_hbm.at[r, c, pl.ds(s * ROWS_PER_TILE, ROWS_PER_TILE)],
        )


@functools.lru_cache(maxsize=None)
def _sc_kernels():
    mesh = plsc.VectorSubcoreMesh(
        core_axis_name="c", subcore_axis_name="s",
        num_cores=NC, num_subcores=NS,
    )
    params = pltpu.CompilerParams(needs_layout_passes=False)
    deg = pl.kernel(
        _deg_body,
        out_type=jax.ShapeDtypeStruct((NC, 3, 2, N_PAD), jnp.float32),
        mesh=mesh,
        compiler_params=params,
        scratch_types=[
            pltpu.VMEM((NCHUNK, 128), jnp.int32),
            pltpu.VMEM((N_PAD,), jnp.float32),
            pltpu.VMEM((ROWS_PER_TILE,), jnp.float32),
            pltpu.VMEM((ROWS_PER_TILE,), jnp.float32),
            pltpu.VMEM_SHARED((NS, N_PAD), jnp.float32),
        ],
    )
    agg = pl.kernel(
        _agg_body,
        out_type=jax.ShapeDtypeStruct((3, NC, N_PAD, D), jnp.float32),
        mesh=mesh,
        compiler_params=params,
        scratch_types=[
            pltpu.VMEM((NCHUNK, 128), jnp.int32),
            pltpu.VMEM((NCHUNK, 128), jnp.int32),
            pltpu.VMEM((128, D), jnp.float32),
            pltpu.VMEM((64, D), jnp.float32),
            pltpu.VMEM_SHARED((N_PAD, D), jnp.float32),
            pltpu.SemaphoreType.DMA,
        ],
    )
    return deg, agg


def _scale_body(c_ref, o_ref):
    o_ref[...] = lax.rsqrt(1.0 + c_ref[0] + c_ref[1])


def _deg_scales(cnt):
    c5 = cnt.reshape(NC, 3, 2, N_PAD // 128, 128)
    out = pl.pallas_call(
        _scale_body,
        out_shape=jax.ShapeDtypeStruct((3, 2, N_PAD // 128, 128), jnp.float32),
    )(c5)
    return out.reshape(3, 2, N_PAD)


def _mm_body(x_ref, w_ref, d_ref, z_ref):
    t = jnp.dot(x_ref[...], w_ref[0], preferred_element_type=jnp.float32)
    z_ref[...] = (t * d_ref[0])[None]


def _mm_scale(x, W, dout):
    blk = 256
    return pl.pallas_call(
        _mm_body,
        grid=(3, N_PAD // blk),
        in_specs=[
            pl.BlockSpec((blk, D), lambda r, i: (i, 0)),
            pl.BlockSpec((1, D, D), lambda r, i: (r, 0, 0)),
            pl.BlockSpec((1, blk, 1), lambda r, i: (r, i, 0)),
        ],
        out_specs=pl.BlockSpec((1, blk, D), lambda r, i: (r, i, 0)),
        out_shape=jax.ShapeDtypeStruct((3, N_PAD, D), jnp.float32),
    )(x, W, dout)


def _make_combine(relu):
    def body(aggp_ref, z_ref, din_ref, b_ref, o_ref):
        total = jnp.zeros_like(o_ref)
        for r in range(3):
            total += (aggp_ref[r, 0] + aggp_ref[r, 1] + z_ref[r]) * din_ref[r]
        res = total + b_ref[...]
        o_ref[...] = jnp.maximum(res, 0.0) if relu else res

    blk = 256

    def combine(aggp, z, din, bsum):
        return pl.pallas_call(
            body,
            grid=(N_PAD // blk,),
            in_specs=[
                pl.BlockSpec((3, NC, blk, D), lambda i: (0, 0, i, 0)),
                pl.BlockSpec((3, blk, D), lambda i: (0, i, 0)),
                pl.BlockSpec((3, blk, 1), lambda i: (0, i, 0)),
                pl.BlockSpec((1, D), lambda i: (0, 0)),
            ],
            out_specs=pl.BlockSpec((blk, D), lambda i: (i, 0)),
            out_shape=jax.ShapeDtypeStruct((N_PAD, D), jnp.float32),
        )(aggp, z, din, bsum)

    return combine


_combine_relu = _make_combine(True)
_combine_plain = _make_combine(False)


def kernel(x, edge_index_r0, edge_index_r1, edge_index_r2,
           W1_r0, b1_r0, W2_r0, b2_r0,
           W1_r1, b1_r1, W2_r1, b2_r1,
           W1_r2, b1_r2, W2_r2, b2_r2):
    eis = [edge_index_r0, edge_index_r1, edge_index_r2]
    pad = jnp.full((3, E_PAD - E), N, jnp.int32)
    src = jnp.concatenate(
        [jnp.stack([e[0].astype(jnp.int32) for e in eis]), pad], axis=1
    ).reshape(3, NW, NCHUNK, 128)
    dst = jnp.concatenate(
        [jnp.stack([e[1].astype(jnp.int32) for e in eis]), pad], axis=1
    ).reshape(3, NW, NCHUNK, 128)

    x_pad = jnp.concatenate(
        [x, jnp.zeros((N_PAD - N, D), jnp.float32)], axis=0
    )
    W1 = jnp.stack([W1_r0, W1_r1, W1_r2])
    W2 = jnp.stack([W2_r0, W2_r1, W2_r2])
    b1sum = (b1_r0 + b1_r1 + b1_r2).reshape(1, D)
    b2sum = (b2_r0 + b2_r1 + b2_r2).reshape(1, D)

    deg_k, agg_k = _sc_kernels()
    cnt = deg_k(src, dst)
    dscale = _deg_scales(cnt)
    dout = dscale[:, 0].reshape(3, N_PAD, 1)
    din = dscale[:, 1].reshape(3, N_PAD, 1)

    z1 = _mm_scale(x_pad, W1, dout)
    agg1 = agg_k(z1, src, dst)
    h = _combine_relu(agg1, z1, din, b1sum)

    z2 = _mm_scale(h, W2, dout)
    agg2 = agg_k(z2, src, dst)
    out = _combine_plain(agg2, z2, din, b2sum)
    return out[:N]
